# baseline (speedup 1.0000x reference)
import functools

import jax
import jax.numpy as jnp
from jax import lax
from jax.experimental import pallas as pl
from jax.experimental.pallas import tpu as pltpu
from jax.experimental.pallas import tpu_sc as plsc

DIM = 256
CBSZ = 8192
NTOK = 32768
BM = 2048
BN = 2048
GCH = 128


def _prep_body(cb_ref, cn_ref, cb2_ref):
    cb = cb_ref[...]
    ones = jnp.ones((1, DIM), jnp.float32)
    cn_ref[...] = lax.dot_general(
        ones, cb * cb, (((1,), (1,)), ((), ())),
        preferred_element_type=jnp.float32,
        precision=lax.Precision.HIGHEST)
    cb2_ref[...] = (cb * -2.0).astype(jnp.bfloat16)


def _argmin_body(x_ref, cb2_ref, cn_ref, out_ref):
    x = x_ref[...]
    xb = x.astype(jnp.bfloat16)
    xn = jnp.sum(x * x, axis=1, keepdims=True)
    colf = lax.broadcasted_iota(jnp.int32, (BM, BN), 1).astype(jnp.float32)

    rmin = jnp.full((BM, 1), jnp.inf, jnp.float32)
    rargf = jnp.zeros((BM, 1), jnp.float32)
    for j in range(CBSZ // BN):
        s2 = lax.dot_general(
            xb, cb2_ref[pl.ds(j * BN, BN), :], (((1,), (1,)), ((), ())),
            preferred_element_type=jnp.float32)
        d = xn + s2 + cn_ref[:, pl.ds(j * BN, BN)]
        lmin = jnp.min(d, axis=1, keepdims=True)
        largf = jnp.min(jnp.where(d == lmin, colf, 16384.0),
                        axis=1, keepdims=True) + float(j * BN)
        take = lmin < rmin
        rmin = jnp.where(take, lmin, rmin)
        rargf = jnp.where(take, largf, rargf)
    out_ref[...] = rargf.astype(jnp.int32)


def _prep(codebook):
    return pl.pallas_call(
        _prep_body,
        out_shape=[
            jax.ShapeDtypeStruct((1, CBSZ), jnp.float32),
            jax.ShapeDtypeStruct((CBSZ, DIM), jnp.bfloat16),
        ],
    )(codebook)


def _encode(x2d, cb2, cn, base, ntok):
    blocks = base // BM
    idx2d = pl.pallas_call(
        _argmin_body,
        grid=(ntok // BM,),
        in_specs=[
            pl.BlockSpec((BM, DIM), lambda i: (i + blocks, 0)),
            pl.BlockSpec((CBSZ, DIM), lambda i: (0, 0)),
            pl.BlockSpec((1, CBSZ), lambda i: (0, 0)),
        ],
        out_specs=pl.BlockSpec((BM, 1), lambda i: (i, 0)),
        out_shape=jax.ShapeDtypeStruct((ntok, 1), jnp.int32),
        compiler_params=pltpu.CompilerParams(
            dimension_semantics=("parallel",)),
    )(x2d, cb2, cn)
    return idx2d.reshape(-1)


def _gather_body(nc, bpw, table_hbm, idx_hbm, out_hbm, idx_v, rows_v, sem):
    wid = lax.axis_index("s") * nc + lax.axis_index("c")

    def step(c, carry):
        base = wid * bpw + c * GCH
        pltpu.sync_copy(idx_hbm.at[pl.ds(base, GCH)], idx_v)
        pltpu.async_copy(table_hbm.at[idx_v], rows_v, sem).wait()
        pltpu.sync_copy(rows_v, out_hbm.at[pl.ds(base, GCH)])
        return carry

    lax.fori_loop(0, bpw // GCH, step, 0)


def _gather(codebook, idx, ntok):
    info = plsc.get_sparse_core_info()
    nc, ns = info.num_cores, info.num_subcores
    bpw = ntok // (nc * ns)
    mesh = plsc.VectorSubcoreMesh(core_axis_name="c", subcore_axis_name="s")
    f = pl.kernel(
        functools.partial(_gather_body, nc, bpw),
        mesh=mesh,
        out_type=jax.ShapeDtypeStruct((ntok, DIM), jnp.float32),
        scratch_types=[
            pltpu.VMEM((GCH,), jnp.int32),
            pltpu.VMEM((GCH, DIM), jnp.float32),
            pltpu.SemaphoreType.DMA,
        ],
    )
    return f(codebook, idx)


def kernel(x, codebook):
    x2d = x.reshape(-1, DIM)
    cn, cb2 = _prep(codebook)
    half = NTOK // 2
    idx0 = _encode(x2d, cb2, cn, 0, half)
    q0 = _gather(codebook, idx0, half)
    idx1 = _encode(x2d, cb2, cn, half, half)
    q1 = _gather(codebook, idx1, half)
    return jnp.concatenate([q0, q1], axis=0).reshape(x.shape)


# speedup vs baseline: 1.0765x; 1.0765x over previous
import functools

import jax
import jax.numpy as jnp
from jax import lax
from jax.experimental import pallas as pl
from jax.experimental.pallas import tpu as pltpu
from jax.experimental.pallas import tpu_sc as plsc

DIM = 256
CBSZ = 8192
NTOK = 32768
BM = 2048
BN = 2048
GCH = 128


def _prep_body(cb_ref, cn_ref, cb2_ref):
    cb = cb_ref[...]
    ones = jnp.ones((1, DIM), jnp.float32)
    cn_ref[...] = lax.dot_general(
        ones, cb * cb, (((1,), (1,)), ((), ())),
        preferred_element_type=jnp.float32,
        precision=lax.Precision.HIGHEST)
    cb2_ref[...] = (cb * -2.0).astype(jnp.bfloat16)


def _argmin_body(x_ref, cb2_ref, cn_ref, out_ref):
    x = x_ref[...]
    xb = x.astype(jnp.bfloat16)
    xn = jnp.sum(x * x, axis=1, keepdims=True)
    colf = lax.broadcasted_iota(jnp.int32, (BM, BN), 1).astype(jnp.float32)

    rmin = jnp.full((BM, 1), jnp.inf, jnp.float32)
    rargf = jnp.zeros((BM, 1), jnp.float32)
    for j in range(CBSZ // BN):
        s2 = lax.dot_general(
            xb, cb2_ref[pl.ds(j * BN, BN), :], (((1,), (1,)), ((), ())),
            preferred_element_type=jnp.float32)
        d = xn + s2 + cn_ref[:, pl.ds(j * BN, BN)]
        lmin = jnp.min(d, axis=1, keepdims=True)
        largf = jnp.min(jnp.where(d == lmin, colf, 16384.0),
                        axis=1, keepdims=True) + float(j * BN)
        take = lmin < rmin
        rmin = jnp.where(take, lmin, rmin)
        rargf = jnp.where(take, largf, rargf)
    out_ref[...] = rargf.astype(jnp.int32)


def _prep(codebook):
    return pl.pallas_call(
        _prep_body,
        out_shape=[
            jax.ShapeDtypeStruct((1, CBSZ), jnp.float32),
            jax.ShapeDtypeStruct((CBSZ, DIM), jnp.bfloat16),
        ],
    )(codebook)


def _encode(x2d, cb2, cn, base, ntok):
    blocks = base // BM
    idx2d = pl.pallas_call(
        _argmin_body,
        grid=(ntok // BM,),
        in_specs=[
            pl.BlockSpec((BM, DIM), lambda i: (i + blocks, 0)),
            pl.BlockSpec((CBSZ, DIM), lambda i: (0, 0)),
            pl.BlockSpec((1, CBSZ), lambda i: (0, 0)),
        ],
        out_specs=pl.BlockSpec((BM, 1), lambda i: (i, 0)),
        out_shape=jax.ShapeDtypeStruct((ntok, 1), jnp.int32),
        compiler_params=pltpu.CompilerParams(
            dimension_semantics=("parallel",)),
    )(x2d, cb2, cn)
    return idx2d.reshape(-1)


def _gather_body(nc, bpw, table_hbm, idx_hbm, out_hbm, idx_v, rows_v, sem):
    wid = lax.axis_index("s") * nc + lax.axis_index("c")

    def step(c, carry):
        base = wid * bpw + c * GCH
        pltpu.sync_copy(idx_hbm.at[pl.ds(base, GCH)], idx_v)
        pltpu.async_copy(table_hbm.at[idx_v], rows_v, sem).wait()
        pltpu.sync_copy(rows_v, out_hbm.at[pl.ds(base, GCH)])
        return carry

    lax.fori_loop(0, bpw // GCH, step, 0)


def _gather(codebook, idx, ntok):
    info = plsc.get_sparse_core_info()
    nc, ns = info.num_cores, info.num_subcores
    bpw = ntok // (nc * ns)
    mesh = plsc.VectorSubcoreMesh(core_axis_name="c", subcore_axis_name="s")
    f = pl.kernel(
        functools.partial(_gather_body, nc, bpw),
        mesh=mesh,
        out_type=jax.ShapeDtypeStruct((ntok, DIM), jnp.float32),
        scratch_types=[
            pltpu.VMEM((GCH,), jnp.int32),
            pltpu.VMEM((GCH, DIM), jnp.float32),
            pltpu.SemaphoreType.DMA,
        ],
    )
    return f(codebook, idx)


def kernel(x, codebook):
    x2d = x.reshape(-1, DIM)
    cn, cb2 = _prep(codebook)
    idx = _encode(x2d, cb2, cn, 0, NTOK)
    q = _gather(codebook, idx, NTOK)
    return q.reshape(x.shape)
